# Initial kernel scaffold; baseline (speedup 1.0000x reference)
import jax
import jax.numpy as jnp
from jax.experimental import pallas as pl


def kernel(pred):
    raise NotImplementedError("write your pallas kernel here")


# speedup vs baseline: 20.1595x; 20.1595x over previous
import jax
import jax.numpy as jnp
from jax.experimental import pallas as pl
from jax.experimental.pallas import tpu as pltpu

_CONF_THRES = 0.4
_IOU_THRES = 0.45
_MAX_DET = 1000
_MAX_WH = 4096.0
_N = 20000
_NPAD = 20480
_R = _NPAD // 128
_NC = 80

_CH_S = 0
_CH_NX1 = 1
_CH_NY1 = 2
_CH_NX2 = 3
_CH_NY2 = 4
_CH_AREA = 5
_CH_CLS = 6
_CH_BX1 = 7
_CH_BY1 = 8
_CH_BX2 = 9
_CH_BY2 = 10
_CH_LIN = 11
_NCH = 12


def _nms_kernel(pt_ref, out_ref, ch_ref):
    cx = pt_ref[0]
    cy = pt_ref[1]
    w = pt_ref[2]
    h = pt_ref[3]
    obj = pt_ref[4]

    best = pt_ref[5] * obj
    bidx = jnp.zeros((_R, 128), jnp.float32)
    for c in range(1, _NC):
        v = pt_ref[5 + c] * obj
        upd = v > best
        best = jnp.where(upd, v, best)
        bidx = jnp.where(upd, jnp.float32(c), bidx)

    valid = best > _CONF_THRES
    s0 = jnp.where(valid, best, -1e9)

    bx1 = cx - w / 2.0
    by1 = cy - h / 2.0
    bx2 = cx + w / 2.0
    by2 = cy + h / 2.0
    off = bidx * _MAX_WH
    nx1 = bx1 + off
    ny1 = by1 + off
    nx2 = bx2 + off
    ny2 = by2 + off
    areas = (nx2 - nx1) * (ny2 - ny1)

    lin = (jax.lax.broadcasted_iota(jnp.int32, (_R, 128), 0) * 128
           + jax.lax.broadcasted_iota(jnp.int32, (_R, 128), 1)
           ).astype(jnp.float32)

    ch_ref[_CH_S] = s0
    ch_ref[_CH_NX1] = nx1
    ch_ref[_CH_NY1] = ny1
    ch_ref[_CH_NX2] = nx2
    ch_ref[_CH_NY2] = ny2
    ch_ref[_CH_AREA] = areas
    ch_ref[_CH_CLS] = bidx
    ch_ref[_CH_BX1] = bx1
    ch_ref[_CH_BY1] = by1
    ch_ref[_CH_BX2] = bx2
    ch_ref[_CH_BY2] = by2
    ch_ref[_CH_LIN] = lin

    lane = jax.lax.broadcasted_iota(jnp.int32, (1, 128), 1).astype(jnp.float32)

    def body(i, _):
        s = ch_ref[_CH_S]
        linv = ch_ref[_CH_LIN]
        m = jnp.max(s)
        ok = m > -1e8
        r = jnp.where(s >= m, linv, 3.0e7)
        idxf = jnp.min(r)
        ii = idxf.astype(jnp.int32)
        ri = jax.lax.shift_right_logical(ii, 7)
        ci = jnp.float32(0) + (ii & 127).astype(jnp.float32)

        def gat(chan):
            row = ch_ref[chan, pl.ds(ri, 1), :]
            return jnp.sum(jnp.where(lane == ci, row, 0.0))

        gx1 = gat(_CH_BX1)
        gy1 = gat(_CH_BY1)
        gx2 = gat(_CH_BX2)
        gy2 = gat(_CH_BY2)
        gcls = gat(_CH_CLS)

        goff = gcls * _MAX_WH
        bnx1 = gx1 + goff
        bny1 = gy1 + goff
        bnx2 = gx2 + goff
        bny2 = gy2 + goff
        barea = (bnx2 - bnx1) * (bny2 - bny1)

        x1 = jnp.maximum(ch_ref[_CH_NX1], bnx1)
        y1 = jnp.maximum(ch_ref[_CH_NY1], bny1)
        x2 = jnp.minimum(ch_ref[_CH_NX2], bnx2)
        y2 = jnp.minimum(ch_ref[_CH_NY2], bny2)
        inter = jnp.maximum(x2 - x1, 0.0) * jnp.maximum(y2 - y1, 0.0)
        iou = inter / (ch_ref[_CH_AREA] + barea - inter + 1e-9)
        sup = (iou > _IOU_THRES) & ok
        hit = r == idxf
        ch_ref[_CH_S] = jnp.where(sup | hit, -1e9, s)

        okf = jnp.where(ok, 1.0, 0.0)
        row = jnp.where(lane == 0.0, gx1,
              jnp.where(lane == 1.0, gy1,
              jnp.where(lane == 2.0, gx2,
              jnp.where(lane == 3.0, gy2,
              jnp.where(lane == 4.0, m,
              jnp.where(lane == 5.0, gcls, 0.0))))))
        out_ref[pl.ds(i, 1), :] = row * okf
        return 0

    jax.lax.fori_loop(0, _MAX_DET, body, 0)


def kernel(pred):
    p = pred[0]
    pt = jnp.transpose(p)
    pt = jnp.pad(pt, ((0, 0), (0, _NPAD - _N)))
    pt = pt.reshape(85, _R, 128)
    out = pl.pallas_call(
        _nms_kernel,
        out_shape=jax.ShapeDtypeStruct((_MAX_DET, 128), jnp.float32),
        scratch_shapes=[pltpu.VMEM((_NCH, _R, 128), jnp.float32)],
    )(pt)
    return out[:, :6]
